# baseline (speedup 1.0000x reference)
import functools

import jax
import jax.numpy as jnp
from jax import lax
from jax.experimental import pallas as pl
from jax.experimental.pallas import tpu as pltpu
from jax.experimental.pallas import tpu_sc as plsc

BATCH = 16384
LATENT_DIM = 64
NUM_CORES = 2
NUM_SUBCORES = 16
NUM_WORKERS = NUM_CORES * NUM_SUBCORES
B_PER_W = BATCH // NUM_WORKERS

_mesh = plsc.VectorSubcoreMesh(core_axis_name="c", subcore_axis_name="s")


@functools.partial(
    pl.kernel,
    mesh=_mesh,
    compiler_params=pltpu.CompilerParams(use_tc_tiling_on_sc=False),
    out_type=jax.ShapeDtypeStruct((BATCH, LATENT_DIM), jnp.float32),
    scratch_types=[
        pltpu.VMEM((B_PER_W,), jnp.int32),
        pltpu.VMEM((B_PER_W, LATENT_DIM), jnp.float32),
        pltpu.SemaphoreType.DMA,
    ],
)
def _gather_kernel(idx_hbm, table_hbm, out_hbm, idx_v, rows_v, sem):
    wid = lax.axis_index("s") * NUM_CORES + lax.axis_index("c")
    base = wid * B_PER_W
    pltpu.sync_copy(idx_hbm.at[pl.ds(base, B_PER_W)], idx_v)
    pltpu.async_copy(table_hbm.at[idx_v], rows_v, sem).wait()
    pltpu.sync_copy(rows_v, out_hbm.at[pl.ds(base, B_PER_W)])


def kernel(indices, codes):
    return _gather_kernel(indices.astype(jnp.int32), codes)


# speedup vs baseline: 2.2151x; 2.2151x over previous
import functools

import jax
import jax.numpy as jnp
from jax import lax
from jax.experimental import pallas as pl
from jax.experimental.pallas import tpu as pltpu
from jax.experimental.pallas import tpu_sc as plsc

BATCH = 16384
LATENT_DIM = 64
NUM_CORES = 2
NUM_SUBCORES = 16
NUM_WORKERS = NUM_CORES * NUM_SUBCORES
B_PER_W = BATCH // NUM_WORKERS
CHUNK = 32
LANES = 16
GROUPS = 125000

_mesh = plsc.VectorSubcoreMesh(core_axis_name="c", subcore_axis_name="s")


@functools.partial(
    pl.kernel,
    mesh=_mesh,
    out_type=jax.ShapeDtypeStruct((BATCH, LATENT_DIM), jnp.float32),
    scratch_types=[
        pltpu.VMEM((B_PER_W,), jnp.int32),
        pltpu.VMEM((CHUNK, 8, LATENT_DIM), jnp.float32),
        pltpu.VMEM((CHUNK, LATENT_DIM), jnp.float32),
        pltpu.SemaphoreType.DMA,
    ],
)
def _gather_kernel(idx_hbm, table_hbm, out_hbm, idx_v, buf_v, out_v, sem):
    wid = lax.axis_index("s") * NUM_CORES + lax.axis_index("c")
    base = wid * B_PER_W

    pltpu.sync_copy(idx_hbm.at[pl.ds(base, B_PER_W)], idx_v)

    def chunk_body(k, carry):
        c0 = k * CHUNK

        def issue_body(c, carry2):
            b0 = c * LANES
            v16 = idx_v[pl.ds(c0 + b0, LANES)]
            for j in range(LANES):
                g = lax.shift_right_logical(v16[j], 3)
                pltpu.async_copy(table_hbm.at[g], buf_v.at[b0 + j], sem)
            return carry2

        lax.fori_loop(0, CHUNK // LANES, issue_body, 0)

        pltpu.make_async_copy(
            table_hbm.at[pl.ds(0, CHUNK)], buf_v, sem
        ).wait()

        def select_body(c, carry2):
            b0 = c * LANES
            v16 = idx_v[pl.ds(c0 + b0, LANES)]
            for j in range(LANES):
                s = lax.bitwise_and(v16[j], 7)
                b = b0 + j
                for d0 in range(0, LATENT_DIM, LANES):
                    out_v[b, pl.ds(d0, LANES)] = buf_v[b, s, pl.ds(d0, LANES)]
            return carry2

        lax.fori_loop(0, CHUNK // LANES, select_body, 0)

        pltpu.sync_copy(out_v, out_hbm.at[pl.ds(base + c0, CHUNK)])
        return carry

    lax.fori_loop(0, B_PER_W // CHUNK, chunk_body, 0)


def kernel(indices, codes):
    table3 = codes.reshape(GROUPS, 8, LATENT_DIM)
    return _gather_kernel(indices.astype(jnp.int32), table3)


# speedup vs baseline: 2.5610x; 1.1562x over previous
import functools

import jax
import jax.numpy as jnp
from jax import lax
from jax.experimental import pallas as pl
from jax.experimental.pallas import tpu as pltpu
from jax.experimental.pallas import tpu_sc as plsc

BATCH = 16384
LATENT_DIM = 64
NUM_CORES = 2
NUM_SUBCORES = 16
NUM_WORKERS = NUM_CORES * NUM_SUBCORES
B_PER_W = BATCH // NUM_WORKERS
LANES = 16
GROUPS = 125000

_mesh = plsc.VectorSubcoreMesh(core_axis_name="c", subcore_axis_name="s")


@functools.partial(
    pl.kernel,
    mesh=_mesh,
    out_type=jax.ShapeDtypeStruct((BATCH, LATENT_DIM), jnp.float32),
    scratch_types=[
        pltpu.VMEM((B_PER_W,), jnp.int32),
        pltpu.VMEM((B_PER_W, LATENT_DIM), jnp.float32),
        pltpu.SemaphoreType.DMA,
    ],
)
def _gather_kernel(idx_hbm, table_hbm, out_hbm, idx_v, rows_v, sem):
    wid = lax.axis_index("s") * NUM_CORES + lax.axis_index("c")
    base = wid * B_PER_W

    pltpu.sync_copy(idx_hbm.at[pl.ds(base, B_PER_W)], idx_v)

    def issue_body(c, carry):
        b0 = c * LANES
        v16 = idx_v[pl.ds(b0, LANES)]
        for j in range(LANES):
            g = lax.shift_right_logical(v16[j], 3)
            s = lax.bitwise_and(v16[j], 7)
            pltpu.async_copy(table_hbm.at[g, s], rows_v.at[b0 + j], sem)
        return carry

    lax.fori_loop(0, B_PER_W // LANES, issue_body, 0)

    pltpu.make_async_copy(
        out_hbm.at[pl.ds(base, B_PER_W)], rows_v, sem
    ).wait()

    pltpu.sync_copy(rows_v, out_hbm.at[pl.ds(base, B_PER_W)])


def kernel(indices, codes):
    table3 = codes.reshape(GROUPS, 8, LATENT_DIM)
    return _gather_kernel(indices.astype(jnp.int32), table3)
